# baseline (speedup 1.0000x reference)
import functools

import jax
import jax.numpy as jnp
from jax import lax
from jax.experimental import pallas as pl
from jax.experimental.pallas import tpu as pltpu
from jax.experimental.pallas import tpu_sc as plsc

_ROWS_PER_BLOCK = 8
_NUM_SC_WORKERS = 32
_LANES = 16


def _sc_gather_body(c, flat_ref, tgt_ref, tv_out,
                    t_v, idx_v, val_v, tv_v, sem):
    rows_per_w = tv_v.shape[0]
    wid = lax.axis_index("s") * 2 + lax.axis_index("c")
    base = wid * rows_per_w
    for j in range(rows_per_w // _LANES):
        r0 = base + j * _LANES
        pltpu.sync_copy(tgt_ref.at[pl.ds(r0, _LANES)], t_v)
        t = t_v[...]
        rows = r0 + lax.iota(jnp.int32, _LANES)
        idx_v[...] = rows * c + t
        pltpu.async_copy(flat_ref.at[idx_v], val_v, sem).wait()
        tv_v[pl.ds(j * _LANES, _LANES)] = val_v[...]
    pltpu.sync_copy(tv_v, tv_out.at[pl.ds(base, rows_per_w)])


def _sc_gather(flat, tgt, b, c):
    rows_per_w = b // _NUM_SC_WORKERS
    mesh = plsc.VectorSubcoreMesh(core_axis_name="c", subcore_axis_name="s")
    f = pl.kernel(
        functools.partial(_sc_gather_body, c),
        out_type=jax.ShapeDtypeStruct((b,), jnp.float32),
        mesh=mesh,
        scratch_types=[
            pltpu.VMEM((_LANES,), jnp.int32),
            pltpu.VMEM((_LANES,), jnp.int32),
            pltpu.VMEM((_LANES,), jnp.float32),
            pltpu.VMEM((rows_per_w,), jnp.float32),
            pltpu.SemaphoreType.DMA,
        ],
    )
    return f(flat, tgt)


def _count_kernel(x_ref, t_ref, tv_ref, out_ref, *, scale):
    step = pl.program_id(0)

    @pl.when(step == 0)
    def _init():
        out_ref[0, 0] = jnp.float32(0.0)
        out_ref[0, 1] = jnp.float32(0.0)

    x = x_ref[...]
    t = t_ref[...]
    v = tv_ref[...] + jnp.float32(0.0)
    bits = lax.bitcast_convert_type(v, jnp.int32)
    key = jnp.where(bits < 0, bits ^ jnp.int32(0x7FFFFFFF), bits)
    km = key - 1 - (key == 0).astype(jnp.int32)
    bm = jnp.where(km < 0, km ^ jnp.int32(0x7FFFFFFF), km)
    tvm = lax.bitcast_convert_type(bm, jnp.float32)
    rb, cc = x.shape
    col = lax.broadcasted_iota(jnp.int32, (rb, cc), 1)
    thr = jnp.where(col < t, tvm, v)
    rank = jnp.sum((x > thr).astype(jnp.int32), axis=1)
    s = jnp.float32(scale)
    out_ref[0, 0] += jnp.sum((rank < 1).astype(jnp.float32)) * s
    out_ref[0, 1] += jnp.sum((rank < 5).astype(jnp.float32)) * s


def kernel(output, target):
    b, c = output.shape
    rb = _ROWS_PER_BLOCK
    t32 = target.astype(jnp.int32)
    tv = _sc_gather(output.reshape(-1), t32, b, c)
    res = pl.pallas_call(
        functools.partial(_count_kernel, scale=100.0 / b),
        grid=(b // rb,),
        in_specs=[
            pl.BlockSpec((rb, c), lambda i: (i, 0)),
            pl.BlockSpec((rb, 1), lambda i: (i, 0)),
            pl.BlockSpec((rb, 1), lambda i: (i, 0)),
        ],
        out_specs=pl.BlockSpec((1, 2), lambda i: (0, 0), memory_space=pltpu.SMEM),
        out_shape=jax.ShapeDtypeStruct((1, 2), jnp.float32),
    )(output, t32.reshape(b, 1), tv.reshape(b, 1))
    return (res[0, 0], res[0, 1])


# speedup vs baseline: 2.0443x; 2.0443x over previous
import functools

import jax
import jax.numpy as jnp
from jax import lax
from jax.experimental import pallas as pl
from jax.experimental.pallas import tpu as pltpu

_ROWS_PER_BLOCK = 8


def _count_kernel(t_sm, x_ref, t_ref, out_ref, *, scale):
    step = pl.program_id(0)

    @pl.when(step == 0)
    def _init():
        out_ref[0, 0] = jnp.float32(0.0)
        out_ref[0, 1] = jnp.float32(0.0)

    x = x_ref[...]
    rb, cc = x.shape
    base = step * rb
    tvs = []
    for r in range(rb):
        t_r = t_sm[base + r]
        start = pl.multiple_of((t_r // 128) * 128, 128)
        w = x_ref[pl.ds(r, 1), pl.ds(start, 128)]
        lane = lax.broadcasted_iota(jnp.int32, (1, 128), 1) + start
        tvs.append(jnp.sum(jnp.where(lane == t_r, w, jnp.float32(0.0)),
                           axis=1, keepdims=True))
    v0 = jnp.concatenate(tvs, axis=0)
    t = t_ref[...]
    v = v0 + jnp.float32(0.0)
    bits = lax.bitcast_convert_type(v, jnp.int32)
    key = jnp.where(bits < 0, bits ^ jnp.int32(0x7FFFFFFF), bits)
    km = key - 1 - (key == 0).astype(jnp.int32)
    bm = jnp.where(km < 0, km ^ jnp.int32(0x7FFFFFFF), km)
    tvm = lax.bitcast_convert_type(bm, jnp.float32)
    col = lax.broadcasted_iota(jnp.int32, (rb, cc), 1)
    thr = jnp.where(col < t, tvm, v)
    rank = jnp.sum((x > thr).astype(jnp.int32), axis=1)
    s = jnp.float32(scale)
    out_ref[0, 0] += jnp.sum((rank < 1).astype(jnp.float32)) * s
    out_ref[0, 1] += jnp.sum((rank < 5).astype(jnp.float32)) * s


def kernel(output, target):
    b, c = output.shape
    rb = _ROWS_PER_BLOCK
    t32 = target.astype(jnp.int32)
    res = pl.pallas_call(
        functools.partial(_count_kernel, scale=100.0 / b),
        grid_spec=pltpu.PrefetchScalarGridSpec(
            num_scalar_prefetch=1,
            grid=(b // rb,),
            in_specs=[pl.BlockSpec((rb, c), lambda i, t_sm: (i, 0)),
                      pl.BlockSpec((rb, 1), lambda i, t_sm: (i, 0))],
            out_specs=pl.BlockSpec((1, 2), lambda i, t_sm: (0, 0),
                                   memory_space=pltpu.SMEM),
        ),
        out_shape=jax.ShapeDtypeStruct((1, 2), jnp.float32),
    )(t32, output, t32.reshape(b, 1))
    return (res[0, 0], res[0, 1])


# speedup vs baseline: 4.4514x; 2.1774x over previous
import functools

import jax
import jax.numpy as jnp
from jax import lax
from jax.experimental import pallas as pl
from jax.experimental.pallas import tpu as pltpu

_BATCH_PER_STEP = 8
_VOCAB_PER_STEP = 800


def _extract_kernel(t_sm, *refs):
    (*x_refs, out_ref) = refs
    i = pl.program_id(0)
    nb = len(x_refs)
    vals = []
    sub_iota = lax.broadcasted_iota(jnp.int32, (8, 128), 0)
    lane_iota = lax.broadcasted_iota(jnp.int32, (8, 128), 1)
    for k in range(nb):
        b = nb * i + k
        t_b = t_sm[b]
        tile = x_refs[k][...]
        m = (sub_iota == t_b % 8) & (lane_iota == b % 128)
        vals.append(jnp.sum(jnp.where(m, tile, jnp.float32(0.0)),
                            keepdims=True).reshape(1, 1))
    out_ref[...] = jnp.concatenate(vals, axis=0)


def _extract_tv(xt, t32, b):
    nb = _BATCH_PER_STEP
    specs = []
    for k in range(nb):
        specs.append(pl.BlockSpec(
            (8, 128),
            functools.partial(
                lambda kk, i, t_sm: (t_sm[nb * i + kk] // 8, (nb * i + kk) // 128),
                k)))
    return pl.pallas_call(
        _extract_kernel,
        grid_spec=pltpu.PrefetchScalarGridSpec(
            num_scalar_prefetch=1,
            grid=(b // nb,),
            in_specs=specs,
            out_specs=pl.BlockSpec((nb, 1), lambda i, t_sm: (i, 0)),
        ),
        out_shape=jax.ShapeDtypeStruct((b, 1), jnp.float32),
    )(t32, *([xt] * nb))


def _count_kernel(x_ref, t_ref, tv_ref, out_ref, cnt_scr, *, scale):
    step = pl.program_id(0)
    nsteps = pl.num_programs(0)

    @pl.when(step == 0)
    def _init():
        cnt_scr[...] = jnp.zeros_like(cnt_scr)

    x = x_ref[...]
    t = t_ref[...]
    v = tv_ref[...] + jnp.float32(0.0)
    bits = lax.bitcast_convert_type(v, jnp.int32)
    key = jnp.where(bits < 0, bits ^ jnp.int32(0x7FFFFFFF), bits)
    km = key - 1 - (key == 0).astype(jnp.int32)
    bm = jnp.where(km < 0, km ^ jnp.int32(0x7FFFFFFF), km)
    tvm = lax.bitcast_convert_type(bm, jnp.float32)
    vb, bb = x.shape
    col = lax.broadcasted_iota(jnp.int32, (vb, bb), 0) + step * vb
    thr = jnp.where(col < t, tvm, v)
    cnt_scr[...] += jnp.sum((x > thr).astype(jnp.int32), axis=0, keepdims=True)

    @pl.when(step == nsteps - 1)
    def _fin():
        rank = cnt_scr[...]
        s = jnp.float32(scale)
        out_ref[0, 0] = jnp.sum((rank < 1).astype(jnp.float32)) * s
        out_ref[0, 1] = jnp.sum((rank < 5).astype(jnp.float32)) * s


def kernel(output, target):
    b, c = output.shape
    vb = _VOCAB_PER_STEP if c % _VOCAB_PER_STEP == 0 else c
    t32 = target.astype(jnp.int32)
    xt = output.T
    tv = _extract_tv(xt, t32, b)
    res = pl.pallas_call(
        functools.partial(_count_kernel, scale=100.0 / b),
        grid=(c // vb,),
        in_specs=[
            pl.BlockSpec((vb, b), lambda i: (i, 0)),
            pl.BlockSpec((1, b), lambda i: (0, 0)),
            pl.BlockSpec((1, b), lambda i: (0, 0)),
        ],
        out_specs=pl.BlockSpec((1, 2), lambda i: (0, 0), memory_space=pltpu.SMEM),
        out_shape=jax.ShapeDtypeStruct((1, 2), jnp.float32),
        scratch_shapes=[pltpu.VMEM((1, b), jnp.int32)],
    )(xt, t32.reshape(1, b), tv.reshape(1, b))
    return (res[0, 0], res[0, 1])


# speedup vs baseline: 4.9632x; 1.1150x over previous
import functools

import jax
import jax.numpy as jnp
from jax import lax
from jax.experimental import pallas as pl
from jax.experimental.pallas import tpu as pltpu

_BATCH_PER_STEP = 32
_VOCAB_PER_STEP = 800


def _extract_kernel(t_sm, *refs):
    (*x_refs, out_ref) = refs
    i = pl.program_id(0)
    nb = len(x_refs)
    vals = []
    sub_iota = lax.broadcasted_iota(jnp.int32, (8, 128), 0)
    lane_iota = lax.broadcasted_iota(jnp.int32, (8, 128), 1)
    for k in range(nb):
        b = nb * i + k
        t_b = t_sm[b]
        tile = x_refs[k][...]
        m = (sub_iota == t_b % 8) & (lane_iota == b % 128)
        vals.append(jnp.sum(jnp.where(m, tile, jnp.float32(0.0)),
                            keepdims=True).reshape(1, 1))
    out_ref[...] = jnp.concatenate(vals, axis=0)


def _extract_tv(xt, t32, b):
    nb = _BATCH_PER_STEP
    specs = []
    for k in range(nb):
        specs.append(pl.BlockSpec(
            (8, 128),
            functools.partial(
                lambda kk, i, t_sm: (t_sm[nb * i + kk] // 8, (nb * i + kk) // 128),
                k)))
    return pl.pallas_call(
        _extract_kernel,
        grid_spec=pltpu.PrefetchScalarGridSpec(
            num_scalar_prefetch=1,
            grid=(b // nb,),
            in_specs=specs,
            out_specs=pl.BlockSpec((nb, 1), lambda i, t_sm: (i, 0)),
        ),
        out_shape=jax.ShapeDtypeStruct((b, 1), jnp.float32),
    )(t32, *([xt] * nb))


def _count_kernel(x_ref, t_ref, tv_ref, out_ref, cnt_scr, *, scale):
    step = pl.program_id(0)
    nsteps = pl.num_programs(0)

    @pl.when(step == 0)
    def _init():
        cnt_scr[...] = jnp.zeros_like(cnt_scr)

    x = x_ref[...]
    t = t_ref[...]
    v = tv_ref[...] + jnp.float32(0.0)
    bits = lax.bitcast_convert_type(v, jnp.int32)
    key = jnp.where(bits < 0, bits ^ jnp.int32(0x7FFFFFFF), bits)
    km = key - 1 - (key == 0).astype(jnp.int32)
    bm = jnp.where(km < 0, km ^ jnp.int32(0x7FFFFFFF), km)
    tvm = lax.bitcast_convert_type(bm, jnp.float32)
    vb, bb = x.shape
    col = lax.broadcasted_iota(jnp.int32, (vb, bb), 0) + step * vb
    thr = jnp.where(col < t, tvm, v)
    cnt_scr[...] += jnp.sum((x > thr).astype(jnp.int32), axis=0, keepdims=True)

    @pl.when(step == nsteps - 1)
    def _fin():
        rank = cnt_scr[...]
        s = jnp.float32(scale)
        out_ref[0, 0] = jnp.sum((rank < 1).astype(jnp.float32)) * s
        out_ref[0, 1] = jnp.sum((rank < 5).astype(jnp.float32)) * s


def kernel(output, target):
    b, c = output.shape
    vb = _VOCAB_PER_STEP if c % _VOCAB_PER_STEP == 0 else c
    t32 = target.astype(jnp.int32)
    xt = output.T
    tv = _extract_tv(xt, t32, b)
    res = pl.pallas_call(
        functools.partial(_count_kernel, scale=100.0 / b),
        grid=(c // vb,),
        in_specs=[
            pl.BlockSpec((vb, b), lambda i: (i, 0)),
            pl.BlockSpec((1, b), lambda i: (0, 0)),
            pl.BlockSpec((1, b), lambda i: (0, 0)),
        ],
        out_specs=pl.BlockSpec((1, 2), lambda i: (0, 0), memory_space=pltpu.SMEM),
        out_shape=jax.ShapeDtypeStruct((1, 2), jnp.float32),
        scratch_shapes=[pltpu.VMEM((1, b), jnp.int32)],
    )(xt, t32.reshape(1, b), tv.reshape(1, b))
    return (res[0, 0], res[0, 1])


# speedup vs baseline: 6.0390x; 1.2167x over previous
import functools

import jax
import jax.numpy as jnp
from jax import lax
from jax.experimental import pallas as pl
from jax.experimental.pallas import tpu as pltpu

_BATCH_PER_STEP = 64
_VOCAB_PER_STEP = 2000


def _extract_kernel(t_sm, *refs):
    (*x_refs, out_ref) = refs
    i = pl.program_id(0)
    nb = len(x_refs)
    vals = []
    sub_iota = lax.broadcasted_iota(jnp.int32, (8, 128), 0)
    lane_iota = lax.broadcasted_iota(jnp.int32, (8, 128), 1)
    for k in range(nb):
        b = nb * i + k
        t_b = t_sm[b]
        tile = x_refs[k][...]
        m = (sub_iota == t_b % 8) & (lane_iota == b % 128)
        vals.append(jnp.sum(jnp.where(m, tile, jnp.float32(0.0)),
                            keepdims=True).reshape(1, 1))
    out_ref[...] = jnp.concatenate(vals, axis=0)


def _extract_tv(xt, t32, b):
    nb = _BATCH_PER_STEP
    specs = []
    for k in range(nb):
        specs.append(pl.BlockSpec(
            (8, 128),
            functools.partial(
                lambda kk, i, t_sm: (t_sm[nb * i + kk] // 8, (nb * i + kk) // 128),
                k)))
    return pl.pallas_call(
        _extract_kernel,
        grid_spec=pltpu.PrefetchScalarGridSpec(
            num_scalar_prefetch=1,
            grid=(b // nb,),
            in_specs=specs,
            out_specs=pl.BlockSpec((nb, 1), lambda i, t_sm: (i, 0)),
        ),
        out_shape=jax.ShapeDtypeStruct((b, 1), jnp.float32),
    )(t32, *([xt] * nb))


def _count_kernel(x_ref, t_ref, tv_ref, out_ref, cnt_scr, *, scale):
    step = pl.program_id(0)
    nsteps = pl.num_programs(0)

    @pl.when(step == 0)
    def _init():
        cnt_scr[...] = jnp.zeros_like(cnt_scr)

    x = x_ref[...]
    t = t_ref[...]
    v = tv_ref[...] + jnp.float32(0.0)
    bits = lax.bitcast_convert_type(v, jnp.int32)
    key = jnp.where(bits < 0, bits ^ jnp.int32(0x7FFFFFFF), bits)
    km = key - 1 - (key == 0).astype(jnp.int32)
    bm = jnp.where(km < 0, km ^ jnp.int32(0x7FFFFFFF), km)
    tvm = lax.bitcast_convert_type(bm, jnp.float32)
    vb, bb = x.shape
    col = lax.broadcasted_iota(jnp.int32, (vb, bb), 0) + step * vb
    thr = jnp.where(col < t, tvm, v)
    cnt_scr[...] += jnp.sum((x > thr).astype(jnp.int32), axis=0, keepdims=True)

    @pl.when(step == nsteps - 1)
    def _fin():
        rank = cnt_scr[...]
        s = jnp.float32(scale)
        out_ref[0, 0] = jnp.sum((rank < 1).astype(jnp.float32)) * s
        out_ref[0, 1] = jnp.sum((rank < 5).astype(jnp.float32)) * s


def kernel(output, target):
    b, c = output.shape
    vb = _VOCAB_PER_STEP if c % _VOCAB_PER_STEP == 0 else c
    t32 = target.astype(jnp.int32)
    xt = output.T
    tv = _extract_tv(xt, t32, b)
    res = pl.pallas_call(
        functools.partial(_count_kernel, scale=100.0 / b),
        grid=(c // vb,),
        in_specs=[
            pl.BlockSpec((vb, b), lambda i: (i, 0)),
            pl.BlockSpec((1, b), lambda i: (0, 0)),
            pl.BlockSpec((1, b), lambda i: (0, 0)),
        ],
        out_specs=pl.BlockSpec((1, 2), lambda i: (0, 0), memory_space=pltpu.SMEM),
        out_shape=jax.ShapeDtypeStruct((1, 2), jnp.float32),
        scratch_shapes=[pltpu.VMEM((1, b), jnp.int32)],
    )(xt, t32.reshape(1, b), tv.reshape(1, b))
    return (res[0, 0], res[0, 1])


# speedup vs baseline: 6.2618x; 1.0369x over previous
import functools

import jax
import jax.numpy as jnp
from jax import lax
from jax.experimental import pallas as pl
from jax.experimental.pallas import tpu as pltpu

_BATCH_PER_STEP = 64
_VOCAB_PER_STEP = 4000


def _extract_kernel(t_sm, *refs):
    (*x_refs, out_ref) = refs
    i = pl.program_id(0)
    nb = len(x_refs)
    vals = []
    sub_iota = lax.broadcasted_iota(jnp.int32, (8, 128), 0)
    lane_iota = lax.broadcasted_iota(jnp.int32, (8, 128), 1)
    for k in range(nb):
        b = nb * i + k
        t_b = t_sm[b]
        tile = x_refs[k][...]
        m = (sub_iota == t_b % 8) & (lane_iota == b % 128)
        vals.append(jnp.sum(jnp.where(m, tile, jnp.float32(0.0)),
                            keepdims=True).reshape(1, 1))
    out_ref[...] = jnp.concatenate(vals, axis=0)


def _extract_tv(xt, t32, b):
    nb = _BATCH_PER_STEP
    specs = []
    for k in range(nb):
        specs.append(pl.BlockSpec(
            (8, 128),
            functools.partial(
                lambda kk, i, t_sm: (t_sm[nb * i + kk] // 8, (nb * i + kk) // 128),
                k)))
    return pl.pallas_call(
        _extract_kernel,
        grid_spec=pltpu.PrefetchScalarGridSpec(
            num_scalar_prefetch=1,
            grid=(b // nb,),
            in_specs=specs,
            out_specs=pl.BlockSpec((nb, 1), lambda i, t_sm: (i, 0)),
        ),
        out_shape=jax.ShapeDtypeStruct((b, 1), jnp.float32),
    )(t32, *([xt] * nb))


def _count_kernel(x_ref, t_ref, tv_ref, out_ref, cnt_scr, *, scale):
    step = pl.program_id(0)
    nsteps = pl.num_programs(0)

    @pl.when(step == 0)
    def _init():
        cnt_scr[...] = jnp.zeros_like(cnt_scr)

    x = x_ref[...]
    t = t_ref[...]
    v = tv_ref[...] + jnp.float32(0.0)
    bits = lax.bitcast_convert_type(v, jnp.int32)
    key = jnp.where(bits < 0, bits ^ jnp.int32(0x7FFFFFFF), bits)
    km = key - 1 - (key == 0).astype(jnp.int32)
    bm = jnp.where(km < 0, km ^ jnp.int32(0x7FFFFFFF), km)
    tvm = lax.bitcast_convert_type(bm, jnp.float32)
    vb, bb = x.shape
    col = lax.broadcasted_iota(jnp.int32, (vb, bb), 0) + step * vb
    thr = jnp.where(col < t, tvm, v)
    cnt_scr[...] += jnp.sum((x > thr).astype(jnp.int32), axis=0, keepdims=True)

    @pl.when(step == nsteps - 1)
    def _fin():
        rank = cnt_scr[...]
        s = jnp.float32(scale)
        out_ref[0, 0] = jnp.sum((rank < 1).astype(jnp.float32)) * s
        out_ref[0, 1] = jnp.sum((rank < 5).astype(jnp.float32)) * s


def kernel(output, target):
    b, c = output.shape
    vb = _VOCAB_PER_STEP if c % _VOCAB_PER_STEP == 0 else c
    t32 = target.astype(jnp.int32)
    xt = output.T
    tv = _extract_tv(xt, t32, b)
    res = pl.pallas_call(
        functools.partial(_count_kernel, scale=100.0 / b),
        grid=(c // vb,),
        in_specs=[
            pl.BlockSpec((vb, b), lambda i: (i, 0)),
            pl.BlockSpec((1, b), lambda i: (0, 0)),
            pl.BlockSpec((1, b), lambda i: (0, 0)),
        ],
        out_specs=pl.BlockSpec((1, 2), lambda i: (0, 0), memory_space=pltpu.SMEM),
        out_shape=jax.ShapeDtypeStruct((1, 2), jnp.float32),
        scratch_shapes=[pltpu.VMEM((1, b), jnp.int32)],
    )(xt, t32.reshape(1, b), tv.reshape(1, b))
    return (res[0, 0], res[0, 1])
